# baseline (speedup 1.0000x reference)
import functools

import jax
import jax.numpy as jnp
from jax import lax
from jax.experimental import pallas as pl
from jax.experimental.pallas import tpu as pltpu
from jax.experimental.pallas import tpu_sc as plsc

EMB_DIM = 128
TIMESTEPS = 1000
BATCH = 16384

_INFO = plsc.get_sparse_core_info()
_NC, _NS = _INFO.num_cores, _INFO.num_subcores
_NW = _NC * _NS
_B_PER_W = BATCH // _NW
_NCHUNK = 8
_CHUNK = _B_PER_W // _NCHUNK


def _fuse_body(table_ref, w_ref, b_ref, out_ref):
    out_ref[...] = (
        lax.dot_general(
            table_ref[...],
            w_ref[...],
            (((1,), (1,)), ((), ())),
            preferred_element_type=jnp.float32,
        )
        + b_ref[...]
    )


def _fuse_table(table, W, b2):
    return pl.pallas_call(
        _fuse_body,
        out_shape=jax.ShapeDtypeStruct((TIMESTEPS, EMB_DIM), jnp.float32),
    )(table, W, b2)


_MESH = plsc.VectorSubcoreMesh(core_axis_name="c", subcore_axis_name="s")


@functools.partial(
    pl.kernel,
    mesh=_MESH,
    out_type=jax.ShapeDtypeStruct((BATCH, EMB_DIM), jnp.float32),
    scratch_types=[
        pltpu.VMEM((_B_PER_W,), jnp.int32),
        pltpu.VMEM((_B_PER_W, EMB_DIM), jnp.float32),
        pltpu.VMEM_SHARED((TIMESTEPS, EMB_DIM), jnp.float32),
        pltpu.SemaphoreType.DMA,
        pltpu.SemaphoreType.DMA,
        pltpu.SemaphoreType.DMA,
    ],
)
def _gather_kernel(
    idx_hbm, fused_hbm, out_hbm, idx_v, rows_v, table_s, gsem, hsem, wsem
):
    wid = lax.axis_index("s") * _NC + lax.axis_index("c")
    base = wid * _B_PER_W
    pltpu.sync_copy(idx_hbm.at[pl.ds(base, _B_PER_W)], idx_v)
    sid = lax.axis_index("s")
    half = 504

    @pl.when(sid == 0)
    def _():
        pltpu.sync_copy(fused_hbm.at[pl.ds(0, half)], table_s.at[pl.ds(0, half)])

    @pl.when(sid == 8)
    def _():
        pltpu.sync_copy(
            fused_hbm.at[pl.ds(half, TIMESTEPS - half)],
            table_s.at[pl.ds(half, TIMESTEPS - half)],
        )

    gathers = [
        pltpu.async_copy(
            fused_hbm.at[idx_v.at[pl.ds(i * _CHUNK, _CHUNK)]],
            rows_v.at[pl.ds(i * _CHUNK, _CHUNK)],
            hsem,
        )
        for i in range(2)
    ]
    plsc.subcore_barrier()
    gathers += [
        pltpu.async_copy(
            table_s.at[idx_v.at[pl.ds(i * _CHUNK, _CHUNK)]],
            rows_v.at[pl.ds(i * _CHUNK, _CHUNK)],
            gsem,
        )
        for i in range(2, _NCHUNK)
    ]
    writes = []
    for i in range(_NCHUNK):
        gathers[i].wait()
        writes.append(
            pltpu.async_copy(
                rows_v.at[pl.ds(i * _CHUNK, _CHUNK)],
                out_hbm.at[pl.ds(base + i * _CHUNK, _CHUNK)],
                wsem,
            )
        )
    for w in writes:
        w.wait()


def kernel(t, table, W, b):
    fused = _fuse_table(table, W, b.reshape(1, EMB_DIM))
    return _gather_kernel(t, fused)


# speedup vs baseline: 1.0467x; 1.0467x over previous
import functools

import jax
import jax.numpy as jnp
from jax import lax
from jax.experimental import pallas as pl
from jax.experimental.pallas import tpu as pltpu
from jax.experimental.pallas import tpu_sc as plsc

EMB_DIM = 128
TIMESTEPS = 1000
BATCH = 16384

_INFO = plsc.get_sparse_core_info()
_NC, _NS = _INFO.num_cores, _INFO.num_subcores
_NW = _NC * _NS
_B_PER_W = BATCH // _NW
_NCHUNK = 8
_CHUNK = _B_PER_W // _NCHUNK


def _fuse_body(table_ref, w_ref, b_ref, out_ref):
    out_ref[...] = (
        lax.dot_general(
            table_ref[...],
            w_ref[...],
            (((1,), (1,)), ((), ())),
            preferred_element_type=jnp.float32,
        )
        + b_ref[...]
    )


def _fuse_table(table, W, b2):
    return pl.pallas_call(
        _fuse_body,
        out_shape=jax.ShapeDtypeStruct((TIMESTEPS, EMB_DIM), jnp.float32),
    )(table, W, b2)


_MESH = plsc.VectorSubcoreMesh(core_axis_name="c", subcore_axis_name="s")


@functools.partial(
    pl.kernel,
    mesh=_MESH,
    out_type=jax.ShapeDtypeStruct((BATCH, EMB_DIM), jnp.float32),
    scratch_types=[
        pltpu.VMEM((_B_PER_W,), jnp.int32),
        pltpu.VMEM((_B_PER_W, EMB_DIM), jnp.float32),
        pltpu.VMEM_SHARED((TIMESTEPS, EMB_DIM), jnp.float32),
        pltpu.SemaphoreType.DMA,
        pltpu.SemaphoreType.DMA,
    ],
)
def _gather_kernel(idx_hbm, fused_hbm, out_hbm, idx_v, rows_v, table_s, gsem, wsem):
    wid = lax.axis_index("s") * _NC + lax.axis_index("c")
    base = wid * _B_PER_W
    pltpu.sync_copy(idx_hbm.at[pl.ds(base, _B_PER_W)], idx_v)
    sid = lax.axis_index("s")
    half = 504

    @pl.when(sid == 0)
    def _():
        pltpu.sync_copy(fused_hbm.at[pl.ds(0, half)], table_s.at[pl.ds(0, half)])

    @pl.when(sid == 8)
    def _():
        pltpu.sync_copy(
            fused_hbm.at[pl.ds(half, TIMESTEPS - half)],
            table_s.at[pl.ds(half, TIMESTEPS - half)],
        )

    plsc.subcore_barrier()
    gathers = [
        pltpu.async_copy(
            table_s.at[idx_v.at[pl.ds(i * _CHUNK, _CHUNK)]],
            rows_v.at[pl.ds(i * _CHUNK, _CHUNK)],
            gsem,
        )
        for i in range(_NCHUNK)
    ]
    writes = []
    for i in range(_NCHUNK):
        gathers[i].wait()
        writes.append(
            pltpu.async_copy(
                rows_v.at[pl.ds(i * _CHUNK, _CHUNK)],
                out_hbm.at[pl.ds(base + i * _CHUNK, _CHUNK)],
                wsem,
            )
        )
    for w in writes:
        w.wait()


def kernel(t, table, W, b):
    fused = _fuse_table(table, W, b.reshape(1, EMB_DIM))
    return _gather_kernel(t, fused)


# speedup vs baseline: 1.0514x; 1.0045x over previous
import functools

import jax
import jax.numpy as jnp
from jax import lax
from jax.experimental import pallas as pl
from jax.experimental.pallas import tpu as pltpu
from jax.experimental.pallas import tpu_sc as plsc

EMB_DIM = 128
TIMESTEPS = 1000
BATCH = 16384

_INFO = plsc.get_sparse_core_info()
_NC, _NS = _INFO.num_cores, _INFO.num_subcores
_NW = _NC * _NS
_B_PER_W = BATCH // _NW
_NCHUNK = 8
_CHUNK = _B_PER_W // _NCHUNK


def _fuse_body(table_ref, w_ref, b_ref, out_ref):
    out_ref[...] = (
        lax.dot_general(
            table_ref[...],
            w_ref[...],
            (((1,), (1,)), ((), ())),
            preferred_element_type=jnp.float32,
        )
        + b_ref[...]
    )


def _fuse_table(table, W, b2):
    return pl.pallas_call(
        _fuse_body,
        out_shape=jax.ShapeDtypeStruct((TIMESTEPS, EMB_DIM), jnp.float32),
    )(table, W, b2)


_MESH = plsc.VectorSubcoreMesh(core_axis_name="c", subcore_axis_name="s")


@functools.partial(
    pl.kernel,
    mesh=_MESH,
    out_type=jax.ShapeDtypeStruct((BATCH, EMB_DIM), jnp.float32),
    scratch_types=[
        pltpu.VMEM((_B_PER_W,), jnp.int32),
        pltpu.VMEM((_B_PER_W, EMB_DIM), jnp.float32),
        pltpu.VMEM_SHARED((TIMESTEPS, EMB_DIM), jnp.float32),
        pltpu.SemaphoreType.DMA,
        pltpu.SemaphoreType.DMA,
    ],
)
def _gather_kernel(idx_hbm, fused_hbm, out_hbm, idx_v, rows_v, table_s, gsem, wsem):
    wid = lax.axis_index("s") * _NC + lax.axis_index("c")
    base = wid * _B_PER_W
    pltpu.sync_copy(idx_hbm.at[pl.ds(base, _B_PER_W)], idx_v)
    sid = lax.axis_index("s")
    for j, (lo, sz) in enumerate([(0, 256), (256, 256), (512, 256), (768, 232)]):

        @pl.when(sid == j * 4)
        def _(lo=lo, sz=sz):
            pltpu.sync_copy(fused_hbm.at[pl.ds(lo, sz)], table_s.at[pl.ds(lo, sz)])

    plsc.subcore_barrier()
    gathers = [
        pltpu.async_copy(
            table_s.at[idx_v.at[pl.ds(i * _CHUNK, _CHUNK)]],
            rows_v.at[pl.ds(i * _CHUNK, _CHUNK)],
            gsem,
        )
        for i in range(_NCHUNK)
    ]
    writes = []
    for i in range(_NCHUNK):
        gathers[i].wait()
        writes.append(
            pltpu.async_copy(
                rows_v.at[pl.ds(i * _CHUNK, _CHUNK)],
                out_hbm.at[pl.ds(base + i * _CHUNK, _CHUNK)],
                wsem,
            )
        )
    for w in writes:
        w.wait()


def kernel(t, table, W, b):
    fused = _fuse_table(table, W, b.reshape(1, EMB_DIM))
    return _gather_kernel(t, fused)


# speedup vs baseline: 1.0515x; 1.0001x over previous
import functools

import jax
import jax.numpy as jnp
from jax import lax
from jax.experimental import pallas as pl
from jax.experimental.pallas import tpu as pltpu
from jax.experimental.pallas import tpu_sc as plsc

EMB_DIM = 128
TIMESTEPS = 1000
BATCH = 16384

_INFO = plsc.get_sparse_core_info()
_NC, _NS = _INFO.num_cores, _INFO.num_subcores
_NW = _NC * _NS
_B_PER_W = BATCH // _NW
_NCHUNK = 16
_CHUNK = _B_PER_W // _NCHUNK


def _fuse_body(table_ref, w_ref, b_ref, out_ref):
    out_ref[...] = (
        lax.dot_general(
            table_ref[...],
            w_ref[...],
            (((1,), (1,)), ((), ())),
            preferred_element_type=jnp.float32,
        )
        + b_ref[...]
    )


def _fuse_table(table, W, b2):
    return pl.pallas_call(
        _fuse_body,
        out_shape=jax.ShapeDtypeStruct((TIMESTEPS, EMB_DIM), jnp.float32),
    )(table, W, b2)


_MESH = plsc.VectorSubcoreMesh(core_axis_name="c", subcore_axis_name="s")


@functools.partial(
    pl.kernel,
    mesh=_MESH,
    out_type=jax.ShapeDtypeStruct((BATCH, EMB_DIM), jnp.float32),
    scratch_types=[
        pltpu.VMEM((_B_PER_W,), jnp.int32),
        pltpu.VMEM((_B_PER_W, EMB_DIM), jnp.float32),
        pltpu.VMEM_SHARED((TIMESTEPS, EMB_DIM), jnp.float32),
        pltpu.SemaphoreType.DMA,
        pltpu.SemaphoreType.DMA,
    ],
)
def _gather_kernel(idx_hbm, fused_hbm, out_hbm, idx_v, rows_v, table_s, gsem, wsem):
    wid = lax.axis_index("s") * _NC + lax.axis_index("c")
    base = wid * _B_PER_W
    pltpu.sync_copy(idx_hbm.at[pl.ds(base, _B_PER_W)], idx_v)
    sid = lax.axis_index("s")
    for j, (lo, sz) in enumerate([(0, 256), (256, 256), (512, 256), (768, 232)]):

        @pl.when(sid == j * 4)
        def _(lo=lo, sz=sz):
            pltpu.sync_copy(fused_hbm.at[pl.ds(lo, sz)], table_s.at[pl.ds(lo, sz)])

    plsc.subcore_barrier()
    gathers = [
        pltpu.async_copy(
            table_s.at[idx_v.at[pl.ds(i * _CHUNK, _CHUNK)]],
            rows_v.at[pl.ds(i * _CHUNK, _CHUNK)],
            gsem,
        )
        for i in range(_NCHUNK)
    ]
    writes = []
    for i in range(_NCHUNK):
        gathers[i].wait()
        writes.append(
            pltpu.async_copy(
                rows_v.at[pl.ds(i * _CHUNK, _CHUNK)],
                out_hbm.at[pl.ds(base + i * _CHUNK, _CHUNK)],
                wsem,
            )
        )
    for w in writes:
        w.wait()


def kernel(t, table, W, b):
    fused = _fuse_table(table, W, b.reshape(1, EMB_DIM))
    return _gather_kernel(t, fused)


# speedup vs baseline: 1.0531x; 1.0016x over previous
import functools

import jax
import jax.numpy as jnp
from jax import lax
from jax.experimental import pallas as pl
from jax.experimental.pallas import tpu as pltpu
from jax.experimental.pallas import tpu_sc as plsc

EMB_DIM = 128
TIMESTEPS = 1000
BATCH = 16384

_INFO = plsc.get_sparse_core_info()
_NC, _NS = _INFO.num_cores, _INFO.num_subcores
_NW = _NC * _NS
_B_PER_W = BATCH // _NW
_NCHUNK = 8
_CHUNK = _B_PER_W // _NCHUNK


def _fuse_body(table_ref, w_ref, b_ref, out_ref):
    out_ref[...] = (
        lax.dot_general(
            table_ref[...],
            w_ref[...],
            (((1,), (1,)), ((), ())),
            preferred_element_type=jnp.float32,
        )
        + b_ref[...]
    )


def _fuse_table(table, W, b2):
    return pl.pallas_call(
        _fuse_body,
        out_shape=jax.ShapeDtypeStruct((TIMESTEPS, EMB_DIM), jnp.float32),
    )(table, W, b2)


_MESH = plsc.VectorSubcoreMesh(core_axis_name="c", subcore_axis_name="s")


@functools.partial(
    pl.kernel,
    mesh=_MESH,
    out_type=jax.ShapeDtypeStruct((BATCH, EMB_DIM), jnp.float32),
    scratch_types=[
        pltpu.VMEM((_B_PER_W,), jnp.int32),
        pltpu.VMEM((_B_PER_W, EMB_DIM), jnp.float32),
        pltpu.VMEM_SHARED((TIMESTEPS, EMB_DIM), jnp.float32),
        pltpu.SemaphoreType.DMA,
        pltpu.SemaphoreType.DMA,
    ],
)
def _gather_kernel(idx_hbm, fused_hbm, out_hbm, idx_v, rows_v, table_s, gsem, wsem):
    wid = lax.axis_index("s") * _NC + lax.axis_index("c")
    base = wid * _B_PER_W
    pltpu.sync_copy(idx_hbm.at[pl.ds(base, _B_PER_W)], idx_v)
    sid = lax.axis_index("s")
    for j, (lo, sz) in enumerate([(0, 256), (256, 256), (512, 256), (768, 232)]):

        @pl.when(sid == j * 4)
        def _(lo=lo, sz=sz):
            pltpu.sync_copy(fused_hbm.at[pl.ds(lo, sz)], table_s.at[pl.ds(lo, sz)])

    plsc.subcore_barrier()
    gathers = [
        pltpu.async_copy(
            table_s.at[idx_v.at[pl.ds(i * _CHUNK, _CHUNK)]],
            rows_v.at[pl.ds(i * _CHUNK, _CHUNK)],
            gsem,
        )
        for i in range(_NCHUNK)
    ]
    writes = []
    for i in range(_NCHUNK):
        gathers[i].wait()
        writes.append(
            pltpu.async_copy(
                rows_v.at[pl.ds(i * _CHUNK, _CHUNK)],
                out_hbm.at[pl.ds(base + i * _CHUNK, _CHUNK)],
                wsem,
            )
        )
    for w in writes:
        w.wait()


def kernel(t, table, W, b):
    fused = _fuse_table(table, W, b.reshape(1, EMB_DIM))
    return _gather_kernel(t, fused)
